# baseline (speedup 1.0000x reference)
import functools

import jax
import jax.numpy as jnp
from jax import lax
from jax.experimental import pallas as pl
from jax.experimental.pallas import tpu as pltpu
from jax.experimental.pallas import tpu_sc as plsc

H = 64
NUM_BITS = 20
BATCH = 16384
V = 2**NUM_BITS

_info = plsc.get_sparse_core_info()
_NC, _NS, _L = _info.num_cores, _info.num_subcores, _info.num_lanes
_NW = _NC * _NS
_BPW = BATCH // _NW
_NG = _BPW // _L
_STREAM = 1024
_NCH = _BPW // _STREAM
_TILE_WORDS = V // 128 * 1024
_NB = H // 8


def _gather_body(states_hbm, emb_hbm, out_hbm, st_v, base_v, idx_v, col_v,
                 sem_out, *sems):
    wid = lax.axis_index("s") * _NC + lax.axis_index("c")
    base = wid * _BPW

    pltpu.sync_copy(states_hbm.at[:, pl.ds(base, _BPW)], st_v)

    def pack_body(g, carry):
        acc = st_v[0, pl.ds(g * _L, _L)]
        for j in range(1, NUM_BITS):
            acc = acc + (st_v[j, pl.ds(g * _L, _L)] << j)
        base_v[pl.ds(g * _L, _L)] = (acc >> 7) * 1024 + (acc & 127)
        return carry

    lax.fori_loop(0, _NG, pack_body, None)

    def make_idx_body(a):
        def idx_body(g, carry):
            tbl = g // 8
            vg = g % 8
            b16 = base_v[pl.ds(tbl * 128 + vg * _L, _L)]
            for s in range(8):
                pos = a * 4096 + tbl * 1024 + s * 128 + vg * _L
                idx_v[pl.ds(pos, _L)] = b16 + (a * _TILE_WORDS + s * 128)
            return carry
        return idx_body

    pending = []
    out_writes = []

    def finish_band(a, copies):
        for c, cp in enumerate(copies):
            cp.wait()
            pos = a * 4096 + c * _STREAM
            out_writes.append(pltpu.async_copy(
                col_v.at[pl.ds(pos, _STREAM)],
                out_hbm.at[pl.ds(a * (BATCH // 128) * 1024 + 4 * wid * 1024
                                 + c * _STREAM, _STREAM)],
                sem_out))

    prev = None
    for a in range(_NB):
        lax.fori_loop(0, _NG, make_idx_body(a), None)
        copies = []
        for q in range(4096 // _STREAM):
            if True:
                pos = a * 4096 + q * _STREAM
                copies.append(pltpu.async_copy(
                    emb_hbm.at[idx_v.at[pl.ds(pos, _STREAM)]],
                    col_v.at[pl.ds(pos, _STREAM)],
                    sems[a],
                ))
        if prev is not None:
            finish_band(*prev)
        prev = (a, copies)
    finish_band(*prev)

    for wr in out_writes:
        wr.wait()


@jax.jit
def kernel(states, emb):
    emb_flat = emb.reshape(V // 128, 128, 8, H // 8)
    emb_flat = emb_flat.transpose(2, 0, 3, 1).reshape(-1)
    mesh = plsc.VectorSubcoreMesh(core_axis_name="c", subcore_axis_name="s")
    gather = functools.partial(
        pl.kernel,
        mesh=mesh,
        out_type=jax.ShapeDtypeStruct((BATCH * H,), jnp.float32),
        scratch_types=[
            pltpu.VMEM((NUM_BITS, _BPW), jnp.int32),
            pltpu.VMEM((_BPW,), jnp.int32),
            pltpu.VMEM((H * _BPW,), jnp.int32),
            pltpu.VMEM((H * _BPW,), jnp.float32),
            pltpu.SemaphoreType.DMA,
        ] + [pltpu.SemaphoreType.DMA] * _NB,
        compiler_params=pltpu.CompilerParams(use_tc_tiling_on_sc=False),
    )(_gather_body)
    out4 = gather(states.T, emb_flat).reshape(_NB, BATCH // 128, 8, 128)
    return out4.transpose(1, 3, 0, 2).reshape(BATCH, H)


# speedup vs baseline: 1.0159x; 1.0159x over previous
import functools

import jax
import jax.numpy as jnp
from jax import lax
from jax.experimental import pallas as pl
from jax.experimental.pallas import tpu as pltpu
from jax.experimental.pallas import tpu_sc as plsc

H = 64
NUM_BITS = 20
BATCH = 16384
V = 2**NUM_BITS

_info = plsc.get_sparse_core_info()
_NC, _NS, _L = _info.num_cores, _info.num_subcores, _info.num_lanes
_NW = _NC * _NS
_BPW = BATCH // _NW
_NG = _BPW // _L
_STREAM = 4096
_NCH = _BPW // _STREAM
_TILE_WORDS = V // 128 * 1024
_NB = H // 8


def _gather_body(states_hbm, emb_hbm, out_hbm, st_v, base_v, idx_v, col_v,
                 sem_out, *sems):
    wid = lax.axis_index("s") * _NC + lax.axis_index("c")
    base = wid * _BPW

    pltpu.sync_copy(states_hbm.at[:, pl.ds(base, _BPW)], st_v)

    def pack_body(g, carry):
        acc = st_v[0, pl.ds(g * _L, _L)]
        for j in range(1, NUM_BITS):
            acc = acc + (st_v[j, pl.ds(g * _L, _L)] << j)
        base_v[pl.ds(g * _L, _L)] = (acc >> 7) * 1024 + (acc & 127)
        return carry

    lax.fori_loop(0, _NG, pack_body, None)

    def make_idx_body(a):
        def idx_body(g, carry):
            tbl = g // 8
            vg = g % 8
            b16 = base_v[pl.ds(tbl * 128 + vg * _L, _L)]
            for s in range(8):
                pos = a * 4096 + tbl * 1024 + s * 128 + vg * _L
                idx_v[pl.ds(pos, _L)] = b16 + (a * _TILE_WORDS + s * 128)
            return carry
        return idx_body

    out_writes = []

    def finish_band(a, copies):
        for cp in copies:
            cp.wait()
        out_writes.append(pltpu.async_copy(
            col_v.at[pl.ds(a * 4096, 4096)],
            out_hbm.at[pl.ds((a * (BATCH // 128) + 4 * wid) * 1024, 4096)],
            sem_out))

    prev = None
    for a in range(_NB):
        lax.fori_loop(0, _NG, make_idx_body(a), None)
        copies = []
        for q in range(8 * _BPW // _STREAM):
            pos = a * 4096 + q * _STREAM
            copies.append(pltpu.async_copy(
                emb_hbm.at[idx_v.at[pl.ds(pos, _STREAM)]],
                col_v.at[pl.ds(pos, _STREAM)],
                sems[a],
            ))
        if prev is not None:
            finish_band(*prev)
        prev = (a, copies)
    finish_band(*prev)

    for wr in out_writes:
        wr.wait()


@jax.jit
def kernel(states, emb):
    emb_flat = emb.reshape(V // 128, 128, 8, H // 8)
    emb_flat = emb_flat.transpose(2, 0, 3, 1).reshape(-1)
    mesh = plsc.VectorSubcoreMesh(core_axis_name="c", subcore_axis_name="s")
    gather = functools.partial(
        pl.kernel,
        mesh=mesh,
        out_type=jax.ShapeDtypeStruct((BATCH * H,), jnp.float32),
        scratch_types=[
            pltpu.VMEM((NUM_BITS, _BPW), jnp.int32),
            pltpu.VMEM((_BPW,), jnp.int32),
            pltpu.VMEM((H * _BPW,), jnp.int32),
            pltpu.VMEM((H * _BPW,), jnp.float32),
            pltpu.SemaphoreType.DMA,
        ] + [pltpu.SemaphoreType.DMA] * _NB,
        compiler_params=pltpu.CompilerParams(use_tc_tiling_on_sc=False),
    )(_gather_body)
    out4 = gather(states.T, emb_flat).reshape(_NB, BATCH // 128, 8, 128)
    return out4.transpose(1, 3, 0, 2).reshape(BATCH, H)
